# Initial kernel scaffold; baseline (speedup 1.0000x reference)
import jax
import jax.numpy as jnp
from jax.experimental import pallas as pl


def kernel(input, timestamp, train, W, b):
    raise NotImplementedError("write your pallas kernel here")


# speedup vs baseline: 6.3422x; 6.3422x over previous
import functools

import jax
import jax.numpy as jnp
from jax import lax
from jax.experimental import pallas as pl
from jax.experimental.pallas import tpu as pltpu
from jax.experimental.pallas import tpu_sc as plsc

_N_TIME = 1000
_OUT_DIM = 8
_PER_TIME = 1.0 / 1000.0
_LANES = 16
_CHUNK = 128


@functools.lru_cache(maxsize=None)
def _build(BL):
    info = plsc.get_sparse_core_info()
    nc, ns = info.num_cores, info.num_subcores
    nw = nc * ns
    assert BL % nw == 0 and (BL // nw) % 8 == 0
    n_per = BL // nw
    n_chunks = -(-n_per // _CHUNK)
    n_pad = n_chunks * _CHUNK

    mesh = plsc.VectorSubcoreMesh(core_axis_name="c", subcore_axis_name="s")

    @functools.partial(
        pl.kernel,
        mesh=mesh,
        out_type=jax.ShapeDtypeStruct((BL, _OUT_DIM), jnp.float32),
        scratch_types=[
            pltpu.VMEM((n_pad,), jnp.float32),
            pltpu.VMEM((n_pad,), jnp.int32),
            pltpu.VMEM((n_pad, _OUT_DIM), jnp.float32),
            pltpu.SemaphoreType.DMA,
        ],
        compiler_params=pltpu.CompilerParams(use_tc_tiling_on_sc=False),
    )
    def gather_kernel(table_hbm, ts_hbm, out_hbm, ts_v, idx_v, rows_v, sem):
        wid = lax.axis_index("s") * nc + lax.axis_index("c")
        base = wid * n_per
        pltpu.sync_copy(ts_hbm.at[pl.ds(base, n_per)], ts_v.at[pl.ds(0, n_per)])
        zeros = jnp.zeros((_LANES,), jnp.float32)
        for m in range(n_per, n_pad, _LANES):
            ts_v[pl.ds(m, _LANES)] = zeros

        def body(i, carry):
            t = ts_v[pl.ds(i * _LANES, _LANES)]
            p = t / jnp.float32(_PER_TIME)
            ix = jnp.minimum(jnp.maximum(p.astype(jnp.int32), 0), _N_TIME - 1)
            idx_v[pl.ds(i * _LANES, _LANES)] = ix
            return carry

        lax.fori_loop(0, n_pad // _LANES, body, 0)

        copies = [
            pltpu.async_copy(
                table_hbm.at[idx_v.at[pl.ds(j * _CHUNK, _CHUNK)]],
                rows_v.at[pl.ds(j * _CHUNK, _CHUNK)],
                sem,
            )
            for j in range(n_chunks)
        ]
        for c in copies:
            c.wait()
        pltpu.sync_copy(rows_v.at[pl.ds(0, n_per)], out_hbm.at[pl.ds(base, n_per)])

    return gather_kernel


def kernel(input, timestamp, train, W, b):
    B, L = input.shape
    ts = timestamp[:, :-1]
    table = W.T + b[None, :]
    BL = B * L
    out = _build(BL)(table, ts.reshape(BL))
    return (out.reshape(B, L, _OUT_DIM), ts)


# speedup vs baseline: 7.8501x; 1.2378x over previous
import functools

import jax
import jax.numpy as jnp
from jax import lax
from jax.experimental import pallas as pl
from jax.experimental.pallas import tpu as pltpu
from jax.experimental.pallas import tpu_sc as plsc

_N_TIME = 1000
_OUT_DIM = 8
_PER_TIME = 1.0 / 1000.0
_LANES = 16


@functools.lru_cache(maxsize=None)
def _build(B, L):
    info = plsc.get_sparse_core_info()
    nc, ns = info.num_cores, info.num_subcores
    nw = nc * ns
    assert B % nw == 0
    rows_w = B // nw
    n_per = rows_w * L
    n_src = rows_w * (L + 1)
    assert n_per % _LANES == 0 and n_src % 8 == 0 and L <= 128

    mesh = plsc.VectorSubcoreMesh(core_axis_name="c", subcore_axis_name="s")

    @functools.partial(
        pl.kernel,
        mesh=mesh,
        out_type=(
            jax.ShapeDtypeStruct((B, L, _OUT_DIM), jnp.float32),
            jax.ShapeDtypeStruct((B, L), jnp.float32),
        ),
        scratch_types=[
            pltpu.VMEM((n_src,), jnp.float32),
            pltpu.VMEM((rows_w, L), jnp.float32),
            pltpu.VMEM((rows_w, L), jnp.int32),
            pltpu.VMEM((rows_w, L, _OUT_DIM), jnp.float32),
            pltpu.SemaphoreType.DMA,
        ],
        compiler_params=pltpu.CompilerParams(
            use_tc_tiling_on_sc=False, needs_layout_passes=False),
    )
    def gather_kernel(table_hbm, ts51_hbm, emb_hbm, ts_hbm,
                      src_v, ts_v, idx_v, rows_v, sem):
        wid = lax.axis_index("s") * nc + lax.axis_index("c")
        r0 = wid * rows_w
        pltpu.sync_copy(ts51_hbm.at[pl.ds(r0 * (L + 1), n_src)], src_v)

        def body(i, carry):
            e = i * _LANES + lax.iota(jnp.int32, _LANES)
            r = lax.div(e, L)
            c = e - r * L
            t = plsc.load_gather(src_v, [e + r])
            p = t / jnp.float32(_PER_TIME)
            ix = jnp.minimum(jnp.maximum(p.astype(jnp.int32), 0), _N_TIME - 1)
            plsc.store_scatter(idx_v, [r, c], ix)
            plsc.store_scatter(ts_v, [r, c], t)
            return carry

        lax.fori_loop(0, n_per // _LANES, body, 0)

        copies = [
            pltpu.async_copy(
                table_hbm.at[idx_v.at[i]],
                rows_v.at[i],
                sem,
            )
            for i in range(rows_w)
        ]
        for cp in copies:
            cp.wait()
        pltpu.sync_copy(rows_v, emb_hbm.at[pl.ds(r0, rows_w)])
        pltpu.sync_copy(ts_v, ts_hbm.at[pl.ds(r0, rows_w)])

    return gather_kernel


def kernel(input, timestamp, train, W, b):
    B, L = input.shape
    table = W.T + b[None, :]
    emb, ts = _build(B, L)(table, timestamp.reshape(B * (L + 1)))
    return (emb, ts)


# speedup vs baseline: 15.9145x; 2.0273x over previous
import functools

import jax
import jax.numpy as jnp
from jax import lax
from jax.experimental import pallas as pl
from jax.experimental.pallas import tpu as pltpu
from jax.experimental.pallas import tpu_sc as plsc

_N_TIME = 1000
_OUT_DIM = 8
_PER_TIME = 1.0 / 1000.0
_LANES = 16


@functools.lru_cache(maxsize=None)
def _build(B, L):
    info = plsc.get_sparse_core_info()
    nc, ns = info.num_cores, info.num_subcores
    nw = nc * ns
    assert B % (nw * _LANES) == 0
    bw = B // nw
    nb = bw // _LANES

    mesh = plsc.VectorSubcoreMesh(core_axis_name="c", subcore_axis_name="s")

    @functools.partial(
        pl.kernel,
        mesh=mesh,
        out_type=(
            jax.ShapeDtypeStruct((L * _OUT_DIM, B), jnp.float32),
            jax.ShapeDtypeStruct((L, B), jnp.float32),
        ),
        scratch_types=[
            pltpu.VMEM((L + 1, bw), jnp.float32),
            pltpu.VMEM((_N_TIME * _OUT_DIM,), jnp.float32),
            pltpu.VMEM((L * _OUT_DIM, bw), jnp.float32),
        ],
        compiler_params=pltpu.CompilerParams(
            use_tc_tiling_on_sc=False, needs_layout_passes=False),
    )
    def gather_kernel(tsT_hbm, table_hbm, embT_hbm, tsoutT_hbm,
                      src_v, w_v, emb_v):
        wid = lax.axis_index("s") * nc + lax.axis_index("c")
        b0 = wid * bw
        pltpu.sync_copy(tsT_hbm.at[:, pl.ds(b0, bw)], src_v)
        pltpu.sync_copy(table_hbm, w_v)
        wrefs = [w_v.at[pl.ds(d * _N_TIME, _N_TIME)] for d in range(_OUT_DIM)]
        lanes = lax.iota(jnp.int32, _LANES)

        def body(i, carry):
            l = lax.div(i, nb)
            k = i - l * nb
            col = k * _LANES + lanes
            t = plsc.load_gather(src_v, [jnp.full((_LANES,), l), col])
            p = t / jnp.float32(_PER_TIME)
            ix = jnp.minimum(jnp.maximum(p.astype(jnp.int32), 0), _N_TIME - 1)
            row0 = l * _OUT_DIM
            for d in range(_OUT_DIM):
                val = plsc.load_gather(wrefs[d], [ix])
                plsc.store_scatter(emb_v, [jnp.full((_LANES,), row0 + d), col], val)
            return carry

        lax.fori_loop(0, L * nb, body, 0)

        pltpu.sync_copy(emb_v, embT_hbm.at[:, pl.ds(b0, bw)])
        pltpu.sync_copy(src_v.at[pl.ds(0, L), :], tsoutT_hbm.at[:, pl.ds(b0, bw)])

    return gather_kernel


def kernel(input, timestamp, train, W, b):
    B, L = input.shape
    table = (W + b[:, None]).reshape(_OUT_DIM * _N_TIME)
    embT, tsT = _build(B, L)(timestamp.T, table)
    emb = embT.reshape(L, _OUT_DIM, B).transpose(2, 0, 1)
    return (emb, tsT.T)


# speedup vs baseline: 15.9510x; 1.0023x over previous
import functools

import jax
import jax.numpy as jnp
from jax import lax
from jax.experimental import pallas as pl
from jax.experimental.pallas import tpu as pltpu
from jax.experimental.pallas import tpu_sc as plsc

_N_TIME = 1000
_OUT_DIM = 8
_PER_TIME = 1.0 / 1000.0
_LANES = 16


@functools.lru_cache(maxsize=None)
def _build(B, L):
    info = plsc.get_sparse_core_info()
    nc, ns = info.num_cores, info.num_subcores
    nw = nc * ns
    assert B % (nw * _LANES) == 0
    bw = B // nw
    nb = bw // _LANES

    mesh = plsc.VectorSubcoreMesh(core_axis_name="c", subcore_axis_name="s")

    @functools.partial(
        pl.kernel,
        mesh=mesh,
        out_type=(
            jax.ShapeDtypeStruct((L * _OUT_DIM, B), jnp.float32),
            jax.ShapeDtypeStruct((L, B), jnp.float32),
        ),
        scratch_types=[
            pltpu.VMEM((L + 1, bw), jnp.float32),
            pltpu.VMEM((_N_TIME * _OUT_DIM,), jnp.float32),
            pltpu.VMEM((L * _OUT_DIM, bw), jnp.float32),
        ],
        compiler_params=pltpu.CompilerParams(
            use_tc_tiling_on_sc=False, needs_layout_passes=False),
    )
    def gather_kernel(tsT_hbm, table_hbm, embT_hbm, tsoutT_hbm,
                      src_v, w_v, emb_v):
        wid = lax.axis_index("s") * nc + lax.axis_index("c")
        b0 = wid * bw
        pltpu.sync_copy(tsT_hbm.at[:, pl.ds(b0, bw)], src_v)
        pltpu.sync_copy(table_hbm, w_v)
        wrefs = [w_v.at[pl.ds(d * _N_TIME, _N_TIME)] for d in range(_OUT_DIM)]

        def body(i, carry):
            l = lax.div(i, nb)
            k = i - l * nb
            c0 = k * _LANES
            t = src_v[l, pl.ds(c0, _LANES)]
            p = t / jnp.float32(_PER_TIME)
            ix = jnp.minimum(jnp.maximum(p.astype(jnp.int32), 0), _N_TIME - 1)
            row0 = l * _OUT_DIM
            for d in range(_OUT_DIM):
                emb_v[row0 + d, pl.ds(c0, _LANES)] = plsc.load_gather(wrefs[d], [ix])
            return carry

        lax.fori_loop(0, L * nb, body, 0, unroll=2)

        pltpu.sync_copy(emb_v, embT_hbm.at[:, pl.ds(b0, bw)])
        pltpu.sync_copy(src_v.at[pl.ds(0, L), :], tsoutT_hbm.at[:, pl.ds(b0, bw)])

    return gather_kernel


def kernel(input, timestamp, train, W, b):
    B, L = input.shape
    table = (W + b[:, None]).reshape(_OUT_DIM * _N_TIME)
    embT, tsT = _build(B, L)(timestamp.T, table)
    emb = embT.reshape(L, _OUT_DIM, B).transpose(2, 0, 1)
    return (emb, tsT.T)
